# baseline (speedup 1.0000x reference)
import functools

import numpy as np

import jax
import jax.numpy as jnp
from jax import lax
from jax.experimental import pallas as pl
from jax.experimental.pallas import tpu as pltpu
from jax.experimental.pallas import tpu_sc as plsc

_TOPK = 2048
_N = 32768
_ROWS = 64
_NC = 2
_NS = 16
_NW = _NC * _NS
_RPW = _ROWS // _NW
_L = 16

_SIGN = np.uint32(0x80000000)


def _to_key(x):
    u = plsc.bitcast(x, jnp.uint32)
    neg = u >= _SIGN
    return jnp.where(neg, ~u, u | _SIGN)


def _key_val(ku):
    pos = ku >= _SIGN
    return plsc.bitcast(jnp.where(pos, ku ^ _SIGN, ~ku), jnp.float32)


@functools.partial(
    pl.kernel,
    out_type=jax.ShapeDtypeStruct((_NW, _L), jnp.float32),
    mesh=plsc.VectorSubcoreMesh(
        core_axis_name="c", subcore_axis_name="s",
        num_cores=_NC, num_subcores=_NS),
    compiler_params=pltpu.CompilerParams(needs_layout_passes=False),
    scratch_types=[
        pltpu.VMEM((_N,), jnp.float32),
        pltpu.VMEM((_N + _L,), jnp.int32),
        pltpu.VMEM((_N + _L,), jnp.int32),
        pltpu.VMEM((_L, _L), jnp.int32),
        pltpu.VMEM((_L,), jnp.float32),
    ],
)
def _sc_topk(scores_hbm, out_hbm, rowbuf, bufa, bufb, hist, outv):
    iota = lax.iota(jnp.int32, _L)
    ones = jnp.ones((_L,), jnp.int32)
    zeros16i = jnp.zeros((_L,), jnp.int32)
    wid = lax.axis_index("s") * _NC + lax.axis_index("c")
    outv[...] = jnp.zeros((_L,), jnp.float32)

    def select(k_rem):
        totals = hist[0]
        for j in range(1, _L):
            totals = totals + hist[j]
        cge = jnp.flip(jnp.cumsum(jnp.flip(totals)))
        pc = plsc.all_reduce_population_count(cge >= k_rem)
        b = jnp.max(pc) - 1
        cnt_gt = jnp.sum(jnp.where(iota > b, totals, 0))
        n_eq = jnp.sum(jnp.where(iota == b, totals, 0))
        return b, cnt_gt, n_eq

    for i in range(_RPW):
        row = wid * _RPW + i
        pltpu.sync_copy(scores_hbm.at[pl.ds(row * _N, _N)], rowbuf)

        for j in range(_L):
            hist[j] = zeros16i

        def h0(ci, carry):
            ku = _to_key(rowbuf[pl.ds(ci * _L, _L)])
            digit = lax.shift_right_logical(ku, jnp.uint32(28)).astype(jnp.int32)
            plsc.addupdate_scatter(hist, [iota, digit], ones)
            return carry

        lax.fori_loop(0, _N // _L, h0, 0)
        b, cnt_gt, n_eq = select(jnp.int32(_TOPK))
        k_rem = jnp.int32(_TOPK) - cnt_gt
        t_bits = b << 28

        def c0(ci, carry):
            offv, sacc = carry
            xv = rowbuf[pl.ds(ci * _L, _L)]
            ku = _to_key(xv)
            digit = lax.shift_right_logical(ku, jnp.uint32(28)).astype(jnp.int32)
            sacc = sacc + jnp.where(digit > b, xv, jnp.float32(0.0))
            eqm = digit == b
            cs = plsc.cumsum(eqm.astype(jnp.int32))
            plsc.store_scatter(bufa, [offv + cs - 1],
                               plsc.bitcast(ku, jnp.int32), mask=eqm)
            offv = offv + plsc.all_reduce_population_count(eqm)
            return offv, sacc

        _, sacc = lax.fori_loop(
            0, _N // _L, c0,
            (jnp.zeros((_L,), jnp.int32), jnp.zeros((_L,), jnp.float32)))
        n = n_eq

        src, dst = bufa, bufb
        for l in range(1, 8):
            sh = jnp.uint32(28 - 4 * l)
            for j in range(_L):
                hist[j] = zeros16i
            nch = (n + (_L - 1)) // _L

            def hl(ci, carry, src=src, sh=sh, n=n):
                base = ci * _L
                ku = plsc.bitcast(src[pl.ds(base, _L)], jnp.uint32)
                digit = (lax.shift_right_logical(ku, sh)
                         & jnp.uint32(15)).astype(jnp.int32)
                valid = (base + iota) < n
                plsc.addupdate_scatter(hist, [iota, digit], ones, mask=valid)
                return carry

            lax.fori_loop(0, nch, hl, 0)
            b, cnt_gt, n_eq = select(k_rem)
            t_bits = t_bits | (b << (28 - 4 * l))
            k_rem = k_rem - cnt_gt

            if l < 7:
                def cl(ci, carry, src=src, dst=dst, sh=sh, n=n, b=b):
                    offv, sacc = carry
                    base = ci * _L
                    kv = src[pl.ds(base, _L)]
                    ku = plsc.bitcast(kv, jnp.uint32)
                    digit = (lax.shift_right_logical(ku, sh)
                             & jnp.uint32(15)).astype(jnp.int32)
                    valid = (base + iota) < n
                    gtm = valid & (digit > b)
                    sacc = sacc + jnp.where(gtm, _key_val(ku), jnp.float32(0.0))
                    eqm = valid & (digit == b)
                    cs = plsc.cumsum(eqm.astype(jnp.int32))
                    plsc.store_scatter(dst, [offv + cs - 1], kv, mask=eqm)
                    offv = offv + plsc.all_reduce_population_count(eqm)
                    return offv, sacc

                _, sacc = lax.fori_loop(
                    0, nch, cl, (jnp.zeros((_L,), jnp.int32), sacc))
                n = n_eq
                src, dst = dst, src

        t_vec = _key_val(plsc.bitcast(jnp.full((_L,), t_bits, jnp.int32),
                                      jnp.uint32))
        mean_vec = (jnp.sum(sacc) + k_rem.astype(jnp.float32) * t_vec) \
            * jnp.float32(1.0 / _TOPK)
        outv[...] = jnp.where(iota == i, mean_vec, outv[...])

    pltpu.sync_copy(outv, out_hbm.at[wid])


def kernel(scores):
    out = _sc_topk(scores.reshape(-1))
    return out[:, :_RPW].reshape(-1)


# speedup vs baseline: 1.1963x; 1.1963x over previous
import functools

import numpy as np

import jax
import jax.numpy as jnp
from jax import lax
from jax.experimental import pallas as pl
from jax.experimental.pallas import tpu as pltpu
from jax.experimental.pallas import tpu_sc as plsc

_TOPK = 2048
_N = 32768
_ROWS = 64
_NC = 2
_NS = 16
_NW = _NC * _NS
_RPW = _ROWS // _NW
_L = 16
_UNROLL = 8

_SIGN = np.uint32(0x80000000)


def _to_key(x):
    u = plsc.bitcast(x, jnp.uint32)
    neg = u >= _SIGN
    return jnp.where(neg, ~u, u | _SIGN)


def _key_val(ku):
    pos = ku >= _SIGN
    return plsc.bitcast(jnp.where(pos, ku ^ _SIGN, ~ku), jnp.float32)


def _digit(ku, sh):
    return (lax.shift_right_logical(ku, jnp.uint32(sh))
            & jnp.uint32(15)).astype(jnp.int32)


@functools.partial(
    pl.kernel,
    out_type=jax.ShapeDtypeStruct((_NW, _L), jnp.float32),
    mesh=plsc.VectorSubcoreMesh(
        core_axis_name="c", subcore_axis_name="s",
        num_cores=_NC, num_subcores=_NS),
    compiler_params=pltpu.CompilerParams(needs_layout_passes=False),
    scratch_types=[
        pltpu.VMEM((_N,), jnp.float32),
        pltpu.VMEM((_N,), jnp.int32),
        pltpu.VMEM((_N,), jnp.int32),
        pltpu.VMEM((_L, _L), jnp.int32),
        pltpu.VMEM((_L,), jnp.float32),
    ],
)
def _sc_topk(scores_hbm, out_hbm, rowbuf, bufa, bufb, hist, outv):
    iota = lax.iota(jnp.int32, _L)
    ones = jnp.ones((_L,), jnp.int32)
    zeros16i = jnp.zeros((_L,), jnp.int32)
    zeros16f = jnp.zeros((_L,), jnp.float32)
    wid = lax.axis_index("s") * _NC + lax.axis_index("c")
    outv[...] = zeros16f

    def select(k_rem):
        totals = hist[0]
        for j in range(1, _L):
            totals = totals + hist[j]
        cge = jnp.flip(jnp.cumsum(jnp.flip(totals)))
        pc = plsc.all_reduce_population_count(cge >= k_rem)
        b = jnp.max(pc) - 1
        cnt_gt = jnp.sum(jnp.where(iota > b, totals, 0))
        return b, cnt_gt

    for i in range(_RPW):
        row = wid * _RPW + i
        pltpu.sync_copy(scores_hbm.at[pl.ds(row * _N, _N)], rowbuf)

        for j in range(_L):
            hist[j] = zeros16i

        def h0(co, carry):
            for u in range(_UNROLL):
                ci = co * _UNROLL + u
                ku = _to_key(rowbuf[pl.ds(ci * _L, _L)])
                plsc.addupdate_scatter(hist, [iota, _digit(ku, 28)], ones)
            return carry

        lax.fori_loop(0, _N // _L // _UNROLL, h0, 0)
        b, cnt_gt = select(jnp.int32(_TOPK))
        k_rem = jnp.int32(_TOPK) - cnt_gt
        t_bits = b << 28

        def c0(co, carry):
            cntv, sacc = carry
            for u in range(_UNROLL):
                ci = co * _UNROLL + u
                xv = rowbuf[pl.ds(ci * _L, _L)]
                ku = _to_key(xv)
                digit = _digit(ku, 28)
                sacc = sacc + jnp.where(digit > b, xv, jnp.float32(0.0))
                eqm = digit == b
                plsc.store_scatter(bufa, [cntv * _L + iota],
                                   plsc.bitcast(ku, jnp.int32), mask=eqm)
                cntv = cntv + eqm.astype(jnp.int32)
            return cntv, sacc

        cntv, sacc = lax.fori_loop(0, _N // _L // _UNROLL, c0,
                                   (zeros16i, zeros16f))

        src, dst = bufa, bufb
        for l in range(1, 8):
            sh = 28 - 4 * l
            for j in range(_L):
                hist[j] = zeros16i
            nco = (jnp.max(cntv) + (_UNROLL - 1)) // _UNROLL

            def hl(co, carry, src=src, sh=sh, cntv=cntv):
                for u in range(_UNROLL):
                    ci = co * _UNROLL + u
                    ku = plsc.bitcast(src[pl.ds(ci * _L, _L)], jnp.uint32)
                    valid = ci < cntv
                    plsc.addupdate_scatter(hist, [iota, _digit(ku, sh)],
                                           ones, mask=valid)
                return carry

            lax.fori_loop(0, nco, hl, 0)
            b, cnt_gt = select(k_rem)
            t_bits = t_bits | (b << sh)
            k_rem = k_rem - cnt_gt

            if l < 7:
                def cl(co, carry, src=src, dst=dst, sh=sh, cntv=cntv, b=b):
                    ncntv, sacc = carry
                    for u in range(_UNROLL):
                        ci = co * _UNROLL + u
                        kv = src[pl.ds(ci * _L, _L)]
                        ku = plsc.bitcast(kv, jnp.uint32)
                        digit = _digit(ku, sh)
                        valid = ci < cntv
                        gtm = valid & (digit > b)
                        sacc = sacc + jnp.where(gtm, _key_val(ku),
                                                jnp.float32(0.0))
                        eqm = valid & (digit == b)
                        plsc.store_scatter(dst, [ncntv * _L + iota],
                                           kv, mask=eqm)
                        ncntv = ncntv + eqm.astype(jnp.int32)
                    return ncntv, sacc

                cntv, sacc = lax.fori_loop(0, nco, cl, (zeros16i, sacc))
                src, dst = dst, src

        t_vec = _key_val(plsc.bitcast(jnp.full((_L,), t_bits, jnp.int32),
                                      jnp.uint32))
        mean_vec = (jnp.sum(sacc) + k_rem.astype(jnp.float32) * t_vec) \
            * jnp.float32(1.0 / _TOPK)
        outv[...] = jnp.where(iota == i, mean_vec, outv[...])

    pltpu.sync_copy(outv, out_hbm.at[wid])


def kernel(scores):
    out = _sc_topk(scores.reshape(-1))
    return out[:, :_RPW].reshape(-1)


# speedup vs baseline: 2.7185x; 2.2724x over previous
import functools

import numpy as np

import jax
import jax.numpy as jnp
from jax import lax
from jax.experimental import pallas as pl
from jax.experimental.pallas import tpu as pltpu
from jax.experimental.pallas import tpu_sc as plsc

_TOPK = 2048
_N = 32768
_ROWS = 64
_NC = 2
_NS = 16
_NW = _NC * _NS
_RPW = _ROWS // _NW
_L = 16
_UNROLL = 8

_SIGN = np.uint32(0x80000000)


def _to_key(x):
    u = plsc.bitcast(x, jnp.uint32)
    neg = u >= _SIGN
    return jnp.where(neg, ~u, u | _SIGN)


def _key_val(ku):
    pos = ku >= _SIGN
    return plsc.bitcast(jnp.where(pos, ku ^ _SIGN, ~ku), jnp.float32)


def _digit(ku, sh):
    return (lax.shift_right_logical(ku, jnp.uint32(sh))
            & jnp.uint32(15)).astype(jnp.int32)


@functools.partial(
    pl.kernel,
    out_type=jax.ShapeDtypeStruct((_NW, _L), jnp.float32),
    mesh=plsc.VectorSubcoreMesh(
        core_axis_name="c", subcore_axis_name="s",
        num_cores=_NC, num_subcores=_NS),
    compiler_params=pltpu.CompilerParams(needs_layout_passes=False),
    scratch_types=[
        pltpu.VMEM((_N,), jnp.float32),
        pltpu.VMEM((_N,), jnp.int32),
        pltpu.VMEM((_N,), jnp.int32),
        pltpu.VMEM((_L, _L), jnp.int32),
        pltpu.VMEM((_L,), jnp.float32),
    ],
)
def _sc_topk(scores_hbm, out_hbm, rowbuf, bufa, bufb, hist, outv):
    iota = lax.iota(jnp.int32, _L)
    ones = jnp.ones((_L,), jnp.int32)
    zeros16i = jnp.zeros((_L,), jnp.int32)
    zeros16f = jnp.zeros((_L,), jnp.float32)
    wid = lax.axis_index("s") * _NC + lax.axis_index("c")
    outv[...] = zeros16f

    def select(k_rem):
        totals = hist[0]
        for j in range(1, _L):
            unskew = (iota + j) & 15
            totals = totals + hist[j].at[unskew].get(
                mode="promise_in_bounds")
        cge = jnp.flip(jnp.cumsum(jnp.flip(totals)))
        pc = plsc.all_reduce_population_count(cge >= k_rem)
        b = jnp.max(pc) - 1
        cnt_gt = jnp.sum(jnp.where(iota > b, totals, 0))
        return b, cnt_gt

    for i in range(_RPW):
        row = wid * _RPW + i
        pltpu.sync_copy(scores_hbm.at[pl.ds(row * _N, _N)], rowbuf)

        for j in range(_L):
            hist[j] = zeros16i

        def h0(ci):
            ku = _to_key(rowbuf[pl.ds(ci * _L, _L)])
            col = (_digit(ku, 28) + iota) & 15
            plsc.addupdate_scatter(hist, [iota, col], ones)

        plsc.parallel_loop(0, _N // _L, unroll=_UNROLL)(h0)
        b, cnt_gt = select(jnp.int32(_TOPK))
        k_rem = jnp.int32(_TOPK) - cnt_gt
        t_bits = b << 28

        def c0(ci, carry):
            cntv, sacc = carry
            xv = rowbuf[pl.ds(ci * _L, _L)]
            ku = _to_key(xv)
            digit = _digit(ku, 28)
            sacc = sacc + jnp.where(digit > b, xv, jnp.float32(0.0))
            eqm = digit == b
            plsc.store_scatter(bufa, [cntv * _L + iota],
                               plsc.bitcast(ku, jnp.int32), mask=eqm)
            return cntv + eqm.astype(jnp.int32), sacc

        cntv, sacc = plsc.parallel_loop(
            0, _N // _L, unroll=_UNROLL, carry=(zeros16i, zeros16f))(c0)

        src, dst = bufa, bufb
        for l in range(1, 8):
            sh = 28 - 4 * l
            for j in range(_L):
                hist[j] = zeros16i
            nch = jnp.max(cntv)

            def hl(ci, src=src, sh=sh, cntv=cntv):
                ku = plsc.bitcast(src[pl.ds(ci * _L, _L)], jnp.uint32)
                col = (_digit(ku, sh) + iota) & 15
                valid = ci < cntv
                plsc.addupdate_scatter(hist, [iota, col], ones, mask=valid)

            plsc.parallel_loop(0, nch, unroll=_UNROLL)(hl)
            b, cnt_gt = select(k_rem)
            t_bits = t_bits | (b << sh)
            k_rem = k_rem - cnt_gt

            if l < 7:
                def cl(ci, carry, src=src, dst=dst, sh=sh, cntv=cntv, b=b):
                    ncntv, sacc = carry
                    kv = src[pl.ds(ci * _L, _L)]
                    ku = plsc.bitcast(kv, jnp.uint32)
                    digit = _digit(ku, sh)
                    valid = ci < cntv
                    gtm = valid & (digit > b)
                    sacc = sacc + jnp.where(gtm, _key_val(ku),
                                            jnp.float32(0.0))
                    eqm = valid & (digit == b)
                    plsc.store_scatter(dst, [ncntv * _L + iota],
                                       kv, mask=eqm)
                    return ncntv + eqm.astype(jnp.int32), sacc

                cntv, sacc = plsc.parallel_loop(
                    0, nch, unroll=_UNROLL, carry=(zeros16i, sacc))(cl)
                src, dst = dst, src

        t_vec = _key_val(plsc.bitcast(jnp.full((_L,), t_bits, jnp.int32),
                                      jnp.uint32))
        mean_vec = (jnp.sum(sacc) + k_rem.astype(jnp.float32) * t_vec) \
            * jnp.float32(1.0 / _TOPK)
        outv[...] = jnp.where(iota == i, mean_vec, outv[...])

    pltpu.sync_copy(outv, out_hbm.at[wid])


def kernel(scores):
    out = _sc_topk(scores.reshape(-1))
    return out[:, :_RPW].reshape(-1)


# speedup vs baseline: 2.7406x; 1.0081x over previous
import functools

import numpy as np

import jax
import jax.numpy as jnp
from jax import lax
from jax.experimental import pallas as pl
from jax.experimental.pallas import tpu as pltpu
from jax.experimental.pallas import tpu_sc as plsc

_TOPK = 2048
_N = 32768
_ROWS = 64
_NC = 2
_NS = 16
_NW = _NC * _NS
_RPW = _ROWS // _NW
_L = 16
_UNROLL = 8
_NQ = 8
_QE = _N // _NQ

_SIGN = np.uint32(0x80000000)


def _to_key(x):
    u = plsc.bitcast(x, jnp.uint32)
    neg = u >= _SIGN
    return jnp.where(neg, ~u, u | _SIGN)


def _key_val(ku):
    pos = ku >= _SIGN
    return plsc.bitcast(jnp.where(pos, ku ^ _SIGN, ~ku), jnp.float32)


def _digit(ku, sh):
    return (lax.shift_right_logical(ku, jnp.uint32(sh))
            & jnp.uint32(15)).astype(jnp.int32)


@functools.partial(
    pl.kernel,
    out_type=jax.ShapeDtypeStruct((_NW, _L), jnp.float32),
    mesh=plsc.VectorSubcoreMesh(
        core_axis_name="c", subcore_axis_name="s",
        num_cores=_NC, num_subcores=_NS),
    compiler_params=pltpu.CompilerParams(needs_layout_passes=False),
    scratch_types=[
        pltpu.VMEM((_N,), jnp.float32),
        pltpu.VMEM((_N,), jnp.int32),
        pltpu.VMEM((_N,), jnp.int32),
        pltpu.VMEM((_L, _L), jnp.int32),
        pltpu.VMEM((_L,), jnp.float32),
        [pltpu.SemaphoreType.DMA] * _NQ,
    ],
)
def _sc_topk(scores_hbm, out_hbm, rowbuf, bufa, bufb, hist, outv, sems):
    iota = lax.iota(jnp.int32, _L)
    ones = jnp.ones((_L,), jnp.int32)
    zeros16i = jnp.zeros((_L,), jnp.int32)
    zeros16f = jnp.zeros((_L,), jnp.float32)
    wid = lax.axis_index("s") * _NC + lax.axis_index("c")
    outv[...] = zeros16f

    def row_copy(r, q):
        return pltpu.make_async_copy(
            scores_hbm.at[pl.ds(r * _N + q * _QE, _QE)],
            rowbuf.at[pl.ds(q * _QE, _QE)],
            sems[q])

    def clear_hist():
        for j in range(_L):
            hist[j] = zeros16i

    def select(k_rem):
        totals = hist[0]
        for j in range(1, _L):
            unskew = (iota + j) & 15
            totals = totals + hist[j].at[unskew].get(
                mode="promise_in_bounds")
        cge = jnp.flip(jnp.cumsum(jnp.flip(totals)))
        pc = plsc.all_reduce_population_count(cge >= k_rem)
        b = jnp.max(pc) - 1
        cnt_gt = jnp.sum(jnp.where(iota > b, totals, 0))
        return b, cnt_gt

    for q in range(_NQ):
        row_copy(wid * _RPW, q).start()

    for i in range(_RPW):
        row = wid * _RPW + i

        clear_hist()
        for q in range(_NQ):
            row_copy(row, q).wait()

            def h0(ci, q=q):
                ku = _to_key(rowbuf[pl.ds((q * _QE // _L + ci) * _L, _L)])
                col = (_digit(ku, 28) + iota) & 15
                plsc.addupdate_scatter(hist, [iota, col], ones)

            plsc.parallel_loop(0, _QE // _L, unroll=_UNROLL)(h0)

        b, cnt_gt = select(jnp.int32(_TOPK))
        k_rem = jnp.int32(_TOPK) - cnt_gt
        t_bits = b << 28

        clear_hist()

        def c0(ci, carry):
            cntv, sacc = carry
            xv = rowbuf[pl.ds(ci * _L, _L)]
            ku = _to_key(xv)
            digit = _digit(ku, 28)
            sacc = sacc + jnp.where(digit > b, xv, jnp.float32(0.0))
            eqm = digit == b
            plsc.store_scatter(bufa, [cntv * _L + iota],
                               plsc.bitcast(ku, jnp.int32), mask=eqm)
            col = (_digit(ku, 24) + iota) & 15
            plsc.addupdate_scatter(hist, [iota, col], ones, mask=eqm)
            return cntv + eqm.astype(jnp.int32), sacc

        cntv, sacc = plsc.parallel_loop(
            0, _N // _L, unroll=_UNROLL, carry=(zeros16i, zeros16f))(c0)

        if i + 1 < _RPW:
            for q in range(_NQ):
                row_copy(row + 1, q).start()

        src, dst = bufa, bufb
        for l in range(1, 8):
            sh = 28 - 4 * l
            b, cnt_gt = select(k_rem)
            t_bits = t_bits | (b << sh)
            k_rem = k_rem - cnt_gt

            if l < 7:
                nch = jnp.max(cntv)
                clear_hist()

                def cl(ci, carry, src=src, dst=dst, sh=sh, cntv=cntv, b=b):
                    ncntv, sacc = carry
                    kv = src[pl.ds(ci * _L, _L)]
                    ku = plsc.bitcast(kv, jnp.uint32)
                    digit = _digit(ku, sh)
                    valid = ci < cntv
                    gtm = valid & (digit > b)
                    sacc = sacc + jnp.where(gtm, _key_val(ku),
                                            jnp.float32(0.0))
                    eqm = valid & (digit == b)
                    plsc.store_scatter(dst, [ncntv * _L + iota],
                                       kv, mask=eqm)
                    col = (_digit(ku, sh - 4) + iota) & 15
                    plsc.addupdate_scatter(hist, [iota, col],
                                           ones, mask=eqm)
                    return ncntv + eqm.astype(jnp.int32), sacc

                cntv, sacc = plsc.parallel_loop(
                    0, nch, unroll=_UNROLL, carry=(zeros16i, sacc))(cl)
                src, dst = dst, src

        t_vec = _key_val(plsc.bitcast(jnp.full((_L,), t_bits, jnp.int32),
                                      jnp.uint32))
        mean_vec = (jnp.sum(sacc) + k_rem.astype(jnp.float32) * t_vec) \
            * jnp.float32(1.0 / _TOPK)
        outv[...] = jnp.where(iota == i, mean_vec, outv[...])

    pltpu.sync_copy(outv, out_hbm.at[wid])


def kernel(scores):
    out = _sc_topk(scores.reshape(-1))
    return out[:, :_RPW].reshape(-1)
